# baseline (speedup 1.0000x reference)
import functools

import jax
import jax.numpy as jnp
from jax import lax
from jax.experimental import pallas as pl
from jax.experimental.pallas import tpu as pltpu
from jax.experimental.pallas import tpu_sc as plsc

T = 2048
D = 1024
FF = 2048
E = 8
M = 256
NB = 24
NPAD = NB * M
NC = 2
NS = 16
NW = NC * NS
RPW = NPAD // NW
TPW = T // NW

_SQRT_2_OVER_PI = 0.7978845608028654
_GELU_C = 0.044715


def _gelu_tanh(h):
    return 0.5 * h * (1.0 + jnp.tanh(_SQRT_2_OVER_PI * (h + _GELU_C * h * h * h)))


def _router_plan_body(x_ref, gw_ref, pos_ref, w_ref, te_ref, bi_ref, ta_ref):
    xv = x_ref[...]
    gwv = gw_ref[...]
    lT = lax.dot_general(gwv, xv, (((1,), (1,)), ((), ())),
                         preferred_element_type=jnp.float32)
    esub = lax.broadcasted_iota(jnp.int32, (E, T), 0)
    m0 = jnp.max(lT, axis=0, keepdims=True)
    e0 = jnp.min(jnp.where(lT == m0, esub, E), axis=0, keepdims=True)
    lT1 = jnp.where(esub == e0, -1e30, lT)
    m1 = jnp.max(lT1, axis=0, keepdims=True)
    e1 = jnp.min(jnp.where(lT1 == m1, esub, E), axis=0, keepdims=True)
    dexp = jnp.exp(m1 - m0)
    w0 = 1.0 / (1.0 + dexp)
    w1c = dexp / (1.0 + dexp)

    h0 = (esub == e0).astype(jnp.float32)
    h1 = (esub == e1).astype(jnp.float32)

    lane = lax.broadcasted_iota(jnp.int32, (E, T), 1)

    def lane_cumsum(a):
        c = a
        s = 1
        while s < T:
            r = pltpu.roll(c, s, 1)
            c = c + jnp.where(lane >= s, r, 0.0)
            s *= 2
        return c

    c0 = lane_cumsum(h0)
    c1 = lane_cumsum(h1)
    cnt0 = jnp.sum(h0, axis=1, keepdims=True)
    cnt = cnt0 + jnp.sum(h1, axis=1, keepdims=True)

    tiles = jnp.floor((cnt + (M - 1)) * (1.0 / M))
    sub = lax.broadcasted_iota(jnp.int32, (E, 1), 0)
    tin = tiles
    s = 1
    while s < E:
        r = pltpu.roll(tin, s, 0)
        tin = tin + jnp.where(sub >= s, r, 0.0)
        s *= 2
    off = (tin - tiles) * M
    endv = tin * M
    nb_used = jnp.sum(tiles, axis=0, keepdims=True)

    p0 = jnp.sum(h0 * (off + c0 - 1.0), axis=0, keepdims=True)
    p1 = jnp.sum(h1 * (off + cnt0 + c1 - 1.0), axis=0, keepdims=True)
    pos_ref[...] = jnp.concatenate([p0, p1], axis=0).astype(jnp.int32)
    w_ref[...] = jnp.concatenate([w0, w1c], axis=0)

    tl = lax.broadcasted_iota(jnp.int32, (1, NB), 1).astype(jnp.float32)
    ts = tl * M
    last = (nb_used - 1.0) * M
    tsc = jnp.minimum(ts, last)
    tej = jnp.sum((tsc >= endv).astype(jnp.float32), axis=0, keepdims=True)
    act = ts < nb_used * M
    te_ref[...] = tej.astype(jnp.int32)
    bi_ref[...] = jnp.where(act, tl, nb_used - 1.0).astype(jnp.int32)
    ta_ref[...] = act.astype(jnp.int32)


def _router_plan(x, gw):
    return pl.pallas_call(
        _router_plan_body,
        out_shape=[
            jax.ShapeDtypeStruct((2, T), jnp.int32),
            jax.ShapeDtypeStruct((2, T), jnp.float32),
            jax.ShapeDtypeStruct((1, NB), jnp.int32),
            jax.ShapeDtypeStruct((1, NB), jnp.int32),
            jax.ShapeDtypeStruct((1, NB), jnp.int32),
        ],
    )(x, gw)


def _plan_scatter_body(pos_hbm, w_hbm, tok_hbm, rw_hbm, pos_v, w_v, tok_v, rw_v):
    c = lax.axis_index("c")
    s = lax.axis_index("s")
    wid = s * NC + c
    lo = wid * RPW
    pltpu.sync_copy(pos_hbm, pos_v)
    pltpu.sync_copy(w_hbm, w_v)
    zi = jnp.zeros((16,), jnp.int32)
    zf = jnp.zeros((16,), jnp.float32)
    for i in range(RPW // 16):
        tok_v[pl.ds(i * 16, 16)] = zi
        rw_v[pl.ds(i * 16, 16)] = zf
    lane = lax.iota(jnp.int32, 16)

    @plsc.parallel_loop(0, (2 * T) // 16, 1, unroll=4)
    def _(i):
        idx = pos_v[pl.ds(i * 16, 16)] - lo
        msk = (idx >= 0) & (idx < RPW)
        pv = lane + i * 16
        tv = pv & (T - 1)
        wv = w_v[pl.ds(i * 16, 16)]
        plsc.store_scatter(tok_v, [idx], tv, mask=msk)
        plsc.store_scatter(rw_v, [idx], wv, mask=msk)

    pltpu.sync_copy(tok_v, tok_hbm.at[pl.ds(lo, RPW)])
    pltpu.sync_copy(rw_v, rw_hbm.at[pl.ds(lo, RPW)])


def _plan_scatter(pos_flat, w_flat):
    mesh = plsc.VectorSubcoreMesh(core_axis_name="c", subcore_axis_name="s",
                                  num_cores=NC, num_subcores=NS)
    return pl.kernel(
        _plan_scatter_body,
        out_type=[
            jax.ShapeDtypeStruct((NPAD,), jnp.int32),
            jax.ShapeDtypeStruct((NPAD,), jnp.float32),
        ],
        mesh=mesh,
        compiler_params=pltpu.CompilerParams(needs_layout_passes=False),
        scratch_types=[
            pltpu.VMEM((2 * T,), jnp.int32),
            pltpu.VMEM((2 * T,), jnp.float32),
            pltpu.VMEM((RPW,), jnp.int32),
            pltpu.VMEM((RPW,), jnp.float32),
        ],
    )(pos_flat, w_flat)


_GCH = 64


def _dispatch_gather_body(tok_hbm, xpk_hbm, xs_hbm, idx_v, buf0, buf1, sem0,
                          sem1):
    c = lax.axis_index("c")
    s = lax.axis_index("s")
    wid = s * NC + c
    base = wid * RPW
    pltpu.sync_copy(tok_hbm.at[pl.ds(base, RPW)], idx_v)
    bufs = (buf0, buf1)
    sems = (sem0, sem1)
    nch = RPW // _GCH
    cps = [pltpu.async_copy(xpk_hbm.at[idx_v.at[pl.ds(0, _GCH)]], buf0, sem0)]
    for ch in range(1, nch):
        cps.append(pltpu.async_copy(
            xpk_hbm.at[idx_v.at[pl.ds(ch * _GCH, _GCH)]], bufs[ch % 2],
            sems[ch % 2]))
        cps[ch - 1].wait()
        pltpu.sync_copy(bufs[(ch - 1) % 2],
                        xs_hbm.at[pl.ds(base + (ch - 1) * _GCH, _GCH)])
    cps[nch - 1].wait()
    pltpu.sync_copy(bufs[(nch - 1) % 2],
                    xs_hbm.at[pl.ds(base + (nch - 1) * _GCH, _GCH)])


def _dispatch_gather(tok, x_pk):
    mesh = plsc.VectorSubcoreMesh(core_axis_name="c", subcore_axis_name="s",
                                  num_cores=NC, num_subcores=NS)
    return pl.kernel(
        _dispatch_gather_body,
        out_type=jax.ShapeDtypeStruct((NPAD, D // 2), jnp.int32),
        mesh=mesh,
        scratch_types=[
            pltpu.VMEM((RPW,), jnp.int32),
            pltpu.VMEM((_GCH, D // 2), jnp.int32),
            pltpu.VMEM((_GCH, D // 2), jnp.int32),
            pltpu.SemaphoreType.DMA,
            pltpu.SemaphoreType.DMA,
        ],
    )(tok, x_pk)


def _ffn_body(te_s, bi_s, ta_s, xs_ref, w1_ref, b1_ref, w2_ref, b2_ref,
              rw_ref, ys_ref):
    i = pl.program_id(0)

    @pl.when(ta_s[i] == 1)
    def _():
        xb = xs_ref[...]
        h = jnp.dot(xb, w1_ref[0], preferred_element_type=jnp.float32)
        h = h + b1_ref[0]
        g = _gelu_tanh(h) * rw_ref[...]
        y = jnp.dot(g.astype(jnp.bfloat16), w2_ref[0],
                    preferred_element_type=jnp.float32)
        ys_ref[...] = y + b2_ref[0]


def _grouped_ffn(te, bi, ta, xs_bf, w1b, b1, w2b, b2, rw_col):
    grid_spec = pltpu.PrefetchScalarGridSpec(
        num_scalar_prefetch=3,
        grid=(NB,),
        in_specs=[
            pl.BlockSpec((M, D), lambda i, te, bi, ta: (bi[i], 0)),
            pl.BlockSpec((1, D, FF), lambda i, te, bi, ta: (te[i], 0, 0)),
            pl.BlockSpec((1, 1, FF), lambda i, te, bi, ta: (te[i], 0, 0)),
            pl.BlockSpec((1, FF, D), lambda i, te, bi, ta: (te[i], 0, 0)),
            pl.BlockSpec((1, 1, D), lambda i, te, bi, ta: (te[i], 0, 0)),
            pl.BlockSpec((M, 1), lambda i, te, bi, ta: (bi[i], 0)),
        ],
        out_specs=pl.BlockSpec((M, D), lambda i, te, bi, ta: (bi[i], 0)),
    )
    return pl.pallas_call(
        _ffn_body,
        grid_spec=grid_spec,
        out_shape=jax.ShapeDtypeStruct((NPAD, D), jnp.float32),
    )(te, bi, ta, xs_bf, w1b, b1, w2b, b2, rw_col)


_CCH = 32


def _combine_body(p0_hbm, p1_hbm, ys_hbm, out_hbm, i0_v, i1_v, bufa,
                  bufb, sema, semb):
    c = lax.axis_index("c")
    s = lax.axis_index("s")
    gbase = (c * NS + s) * TPW
    pltpu.sync_copy(p0_hbm.at[pl.ds(gbase, TPW)], i0_v)
    pltpu.sync_copy(p1_hbm.at[pl.ds(gbase, TPW)], i1_v)
    for rnd in range(TPW // _CCH):
        cb = rnd * _CCH
        cpa = pltpu.async_copy(ys_hbm.at[i0_v.at[pl.ds(cb, _CCH)]], bufa, sema)
        cpb = pltpu.async_copy(ys_hbm.at[i1_v.at[pl.ds(cb, _CCH)]], bufb, semb)
        cpa.wait()
        cpb.wait()

        @plsc.parallel_loop(0, _CCH * (D // 16), 1, unroll=8)
        def _(i):
            r = i >> 6
            sl = pl.ds((i & 63) * 16, 16)
            bufa[r, sl] = bufa[r, sl] + bufb[r, sl]
        pltpu.sync_copy(bufa, out_hbm.at[pl.ds(gbase + cb, _CCH)])


def _combine(pos0, pos1, ys):
    mesh = plsc.VectorSubcoreMesh(core_axis_name="c", subcore_axis_name="s",
                                  num_cores=NC, num_subcores=NS)
    return pl.kernel(
        _combine_body,
        out_type=jax.ShapeDtypeStruct((T, D), jnp.float32),
        mesh=mesh,
        compiler_params=pltpu.CompilerParams(needs_layout_passes=False),
        scratch_types=[
            pltpu.VMEM((TPW,), jnp.int32),
            pltpu.VMEM((TPW,), jnp.int32),
            pltpu.VMEM((_CCH, D), jnp.float32),
            pltpu.VMEM((_CCH, D), jnp.float32),
            pltpu.SemaphoreType.DMA,
            pltpu.SemaphoreType.DMA,
        ],
    )(pos0, pos1, ys)


def kernel(hidden_states, gate_w, w1, b1, w2, b2):
    b, s, d = hidden_states.shape
    x = hidden_states.reshape(T, D)

    pos2, wflat2, te, bi, ta = _router_plan(x, gate_w)
    tok, roww = _plan_scatter(pos2.reshape(2 * T), wflat2.reshape(2 * T))

    x_pk = lax.bitcast_convert_type(
        x.astype(jnp.bfloat16).reshape(T, D // 2, 2), jnp.int32)
    xs_pk = _dispatch_gather(tok, x_pk)
    xs_bf = lax.bitcast_convert_type(xs_pk, jnp.bfloat16).reshape(NPAD, D)

    ys = _grouped_ffn(te.reshape(NB), bi.reshape(NB), ta.reshape(NB),
                      xs_bf, w1.astype(jnp.bfloat16), b1.reshape(E, 1, FF),
                      w2.astype(jnp.bfloat16), b2.reshape(E, 1, D),
                      roww.reshape(NPAD, 1))

    out = _combine(pos2[0], pos2[1], ys)
    return out.reshape(b, s, d)


# speedup vs baseline: 1.0022x; 1.0022x over previous
import functools

import jax
import jax.numpy as jnp
from jax import lax
from jax.experimental import pallas as pl
from jax.experimental.pallas import tpu as pltpu
from jax.experimental.pallas import tpu_sc as plsc

T = 2048
D = 1024
FF = 2048
E = 8
M = 256
NB = 24
NPAD = NB * M
NC = 2
NS = 16
NW = NC * NS
RPW = NPAD // NW
TPW = T // NW

_SQRT_2_OVER_PI = 0.7978845608028654
_GELU_C = 0.044715


def _gelu_tanh(h):
    return 0.5 * h * (1.0 + jnp.tanh(_SQRT_2_OVER_PI * (h + _GELU_C * h * h * h)))


def _router_plan_body(x_ref, gw_ref, pos_ref, w_ref, te_ref, bi_ref, ta_ref):
    xv = x_ref[...]
    gwv = gw_ref[...]
    lT = lax.dot_general(gwv, xv, (((1,), (1,)), ((), ())),
                         preferred_element_type=jnp.float32)
    esub = lax.broadcasted_iota(jnp.int32, (E, T), 0)
    m0 = jnp.max(lT, axis=0, keepdims=True)
    e0 = jnp.min(jnp.where(lT == m0, esub, E), axis=0, keepdims=True)
    lT1 = jnp.where(esub == e0, -1e30, lT)
    m1 = jnp.max(lT1, axis=0, keepdims=True)
    e1 = jnp.min(jnp.where(lT1 == m1, esub, E), axis=0, keepdims=True)
    dexp = jnp.exp(m1 - m0)
    w0 = 1.0 / (1.0 + dexp)
    w1c = dexp / (1.0 + dexp)

    h0 = (esub == e0).astype(jnp.float32)
    h1 = (esub == e1).astype(jnp.float32)

    lane = lax.broadcasted_iota(jnp.int32, (E, T), 1)

    def lane_cumsum(a):
        c = a
        s = 1
        while s < T:
            r = pltpu.roll(c, s, 1)
            c = c + jnp.where(lane >= s, r, 0.0)
            s *= 2
        return c

    c0 = lane_cumsum(h0)
    c1 = lane_cumsum(h1)
    cnt0 = jnp.sum(h0, axis=1, keepdims=True)
    cnt = cnt0 + jnp.sum(h1, axis=1, keepdims=True)

    tiles = jnp.floor((cnt + (M - 1)) * (1.0 / M))
    sub = lax.broadcasted_iota(jnp.int32, (E, 1), 0)
    tin = tiles
    s = 1
    while s < E:
        r = pltpu.roll(tin, s, 0)
        tin = tin + jnp.where(sub >= s, r, 0.0)
        s *= 2
    off = (tin - tiles) * M
    endv = tin * M
    nb_used = jnp.sum(tiles, axis=0, keepdims=True)

    p0 = jnp.sum(h0 * (off + c0 - 1.0), axis=0, keepdims=True)
    p1 = jnp.sum(h1 * (off + cnt0 + c1 - 1.0), axis=0, keepdims=True)
    pos_ref[...] = jnp.concatenate([p0, p1], axis=0).astype(jnp.int32)
    w_ref[...] = jnp.concatenate([w0, w1c], axis=0)

    tl = lax.broadcasted_iota(jnp.int32, (1, NB), 1).astype(jnp.float32)
    ts = tl * M
    last = (nb_used - 1.0) * M
    tsc = jnp.minimum(ts, last)
    tej = jnp.sum((tsc >= endv).astype(jnp.float32), axis=0, keepdims=True)
    act = ts < nb_used * M
    te_ref[...] = tej.astype(jnp.int32)
    bi_ref[...] = jnp.where(act, tl, nb_used - 1.0).astype(jnp.int32)
    ta_ref[...] = act.astype(jnp.int32)


def _router_plan(x, gw):
    return pl.pallas_call(
        _router_plan_body,
        out_shape=[
            jax.ShapeDtypeStruct((2, T), jnp.int32),
            jax.ShapeDtypeStruct((2, T), jnp.float32),
            jax.ShapeDtypeStruct((1, NB), jnp.int32),
            jax.ShapeDtypeStruct((1, NB), jnp.int32),
            jax.ShapeDtypeStruct((1, NB), jnp.int32),
        ],
    )(x, gw)


def _plan_scatter_body(pos_hbm, w_hbm, tok_hbm, rw_hbm, pos_v, w_v, tok_v, rw_v):
    c = lax.axis_index("c")
    s = lax.axis_index("s")
    wid = s * NC + c
    lo = wid * RPW
    pltpu.sync_copy(pos_hbm, pos_v)
    pltpu.sync_copy(w_hbm, w_v)
    zi = jnp.zeros((16,), jnp.int32)
    zf = jnp.zeros((16,), jnp.float32)
    for i in range(RPW // 16):
        tok_v[pl.ds(i * 16, 16)] = zi
        rw_v[pl.ds(i * 16, 16)] = zf
    lane = lax.iota(jnp.int32, 16)

    @plsc.parallel_loop(0, (2 * T) // 16, 1, unroll=4)
    def _(i):
        idx = pos_v[pl.ds(i * 16, 16)] - lo
        msk = (idx >= 0) & (idx < RPW)
        pv = lane + i * 16
        tv = pv & (T - 1)
        wv = w_v[pl.ds(i * 16, 16)]
        plsc.store_scatter(tok_v, [idx], tv, mask=msk)
        plsc.store_scatter(rw_v, [idx], wv, mask=msk)

    pltpu.sync_copy(tok_v, tok_hbm.at[pl.ds(lo, RPW)])
    pltpu.sync_copy(rw_v, rw_hbm.at[pl.ds(lo, RPW)])


def _plan_scatter(pos_flat, w_flat):
    mesh = plsc.VectorSubcoreMesh(core_axis_name="c", subcore_axis_name="s",
                                  num_cores=NC, num_subcores=NS)
    return pl.kernel(
        _plan_scatter_body,
        out_type=[
            jax.ShapeDtypeStruct((NPAD,), jnp.int32),
            jax.ShapeDtypeStruct((NPAD,), jnp.float32),
        ],
        mesh=mesh,
        compiler_params=pltpu.CompilerParams(needs_layout_passes=False),
        scratch_types=[
            pltpu.VMEM((2 * T,), jnp.int32),
            pltpu.VMEM((2 * T,), jnp.float32),
            pltpu.VMEM((RPW,), jnp.int32),
            pltpu.VMEM((RPW,), jnp.float32),
        ],
    )(pos_flat, w_flat)


_GCH = 64


def _dispatch_gather_body(tok_hbm, xpk_hbm, xs_hbm, idx_v, buf0, buf1, sem0,
                          sem1):
    c = lax.axis_index("c")
    s = lax.axis_index("s")
    wid = s * NC + c
    base = wid * RPW
    pltpu.sync_copy(tok_hbm.at[pl.ds(base, RPW)], idx_v)
    bufs = (buf0, buf1)
    sems = (sem0, sem1)
    nch = RPW // _GCH
    cps = [pltpu.async_copy(xpk_hbm.at[idx_v.at[pl.ds(0, _GCH)]], buf0, sem0)]
    for ch in range(1, nch):
        cps.append(pltpu.async_copy(
            xpk_hbm.at[idx_v.at[pl.ds(ch * _GCH, _GCH)]], bufs[ch % 2],
            sems[ch % 2]))
        cps[ch - 1].wait()
        pltpu.sync_copy(bufs[(ch - 1) % 2],
                        xs_hbm.at[pl.ds(base + (ch - 1) * _GCH, _GCH)])
    cps[nch - 1].wait()
    pltpu.sync_copy(bufs[(nch - 1) % 2],
                    xs_hbm.at[pl.ds(base + (nch - 1) * _GCH, _GCH)])


def _dispatch_gather(tok, x_pk):
    mesh = plsc.VectorSubcoreMesh(core_axis_name="c", subcore_axis_name="s",
                                  num_cores=NC, num_subcores=NS)
    return pl.kernel(
        _dispatch_gather_body,
        out_type=jax.ShapeDtypeStruct((NPAD, D // 2), jnp.int32),
        mesh=mesh,
        scratch_types=[
            pltpu.VMEM((RPW,), jnp.int32),
            pltpu.VMEM((_GCH, D // 2), jnp.int32),
            pltpu.VMEM((_GCH, D // 2), jnp.int32),
            pltpu.SemaphoreType.DMA,
            pltpu.SemaphoreType.DMA,
        ],
    )(tok, x_pk)


def _ffn_body(te_s, bi_s, ta_s, xs_ref, w1_ref, b1_ref, w2_ref, b2_ref,
              rw_ref, ys_ref):
    i = pl.program_id(0)

    @pl.when(ta_s[i] == 1)
    def _():
        xb = xs_ref[...]
        h = jnp.dot(xb, w1_ref[0], preferred_element_type=jnp.float32)
        h = h + b1_ref[0]
        g = _gelu_tanh(h) * rw_ref[...]
        y = jnp.dot(g.astype(jnp.bfloat16), w2_ref[0],
                    preferred_element_type=jnp.float32)
        ys_ref[...] = y + b2_ref[0]


def _grouped_ffn(te, bi, ta, xs_bf, w1b, b1, w2b, b2, rw_col):
    grid_spec = pltpu.PrefetchScalarGridSpec(
        num_scalar_prefetch=3,
        grid=(NB,),
        in_specs=[
            pl.BlockSpec((M, D), lambda i, te, bi, ta: (bi[i], 0)),
            pl.BlockSpec((1, D, FF), lambda i, te, bi, ta: (te[i], 0, 0)),
            pl.BlockSpec((1, 1, FF), lambda i, te, bi, ta: (te[i], 0, 0)),
            pl.BlockSpec((1, FF, D), lambda i, te, bi, ta: (te[i], 0, 0)),
            pl.BlockSpec((1, 1, D), lambda i, te, bi, ta: (te[i], 0, 0)),
            pl.BlockSpec((M, 1), lambda i, te, bi, ta: (bi[i], 0)),
        ],
        out_specs=pl.BlockSpec((M, D), lambda i, te, bi, ta: (bi[i], 0)),
    )
    return pl.pallas_call(
        _ffn_body,
        grid_spec=grid_spec,
        out_shape=jax.ShapeDtypeStruct((NPAD, D), jnp.float32),
    )(te, bi, ta, xs_bf, w1b, b1, w2b, b2, rw_col)


_CCH = 16


def _combine_body(p0_hbm, p1_hbm, ys_hbm, out_hbm, i0_v, i1_v, bufa0, bufa1,
                  bufb0, bufb1, sa0, sa1, sb0, sb1):
    c = lax.axis_index("c")
    s = lax.axis_index("s")
    gbase = (c * NS + s) * TPW
    pltpu.sync_copy(p0_hbm.at[pl.ds(gbase, TPW)], i0_v)
    pltpu.sync_copy(p1_hbm.at[pl.ds(gbase, TPW)], i1_v)
    bufa = (bufa0, bufa1)
    bufb = (bufb0, bufb1)
    sas = (sa0, sa1)
    sbs = (sb0, sb1)
    nrnd = TPW // _CCH

    def fire(rnd):
        pb = rnd % 2
        cb = rnd * _CCH
        cpa = pltpu.async_copy(ys_hbm.at[i0_v.at[pl.ds(cb, _CCH)]], bufa[pb],
                               sas[pb])
        cpb = pltpu.async_copy(ys_hbm.at[i1_v.at[pl.ds(cb, _CCH)]], bufb[pb],
                               sbs[pb])
        return cpa, cpb

    cps = fire(0)
    for rnd in range(nrnd):
        pb = rnd % 2
        cpa, cpb = cps
        cpa.wait()
        cpb.wait()
        if rnd + 1 < nrnd:
            cps = fire(rnd + 1)
        a = bufa[pb]
        b = bufb[pb]

        def radd(r, carry):
            for j in range(D // 16):
                sl = slice(j * 16, j * 16 + 16)
                a[r, sl] = a[r, sl] + b[r, sl]
            return carry

        lax.fori_loop(0, _CCH, radd, 0)
        pltpu.sync_copy(a, out_hbm.at[pl.ds(gbase + rnd * _CCH, _CCH)])


def _combine(pos0, pos1, ys):
    mesh = plsc.VectorSubcoreMesh(core_axis_name="c", subcore_axis_name="s",
                                  num_cores=NC, num_subcores=NS)
    return pl.kernel(
        _combine_body,
        out_type=jax.ShapeDtypeStruct((T, D), jnp.float32),
        mesh=mesh,
        compiler_params=pltpu.CompilerParams(needs_layout_passes=False),
        scratch_types=[
            pltpu.VMEM((TPW,), jnp.int32),
            pltpu.VMEM((TPW,), jnp.int32),
            pltpu.VMEM((_CCH, D), jnp.float32),
            pltpu.VMEM((_CCH, D), jnp.float32),
            pltpu.VMEM((_CCH, D), jnp.float32),
            pltpu.VMEM((_CCH, D), jnp.float32),
            pltpu.SemaphoreType.DMA,
            pltpu.SemaphoreType.DMA,
            pltpu.SemaphoreType.DMA,
            pltpu.SemaphoreType.DMA,
        ],
    )(pos0, pos1, ys)


def kernel(hidden_states, gate_w, w1, b1, w2, b2):
    b, s, d = hidden_states.shape
    x = hidden_states.reshape(T, D)

    pos2, wflat2, te, bi, ta = _router_plan(x, gate_w)
    tok, roww = _plan_scatter(pos2.reshape(2 * T), wflat2.reshape(2 * T))

    x_pk = lax.bitcast_convert_type(
        x.astype(jnp.bfloat16).reshape(T, D // 2, 2), jnp.int32)
    xs_pk = _dispatch_gather(tok, x_pk)
    xs_bf = lax.bitcast_convert_type(xs_pk, jnp.bfloat16).reshape(NPAD, D)

    ys = _grouped_ffn(te.reshape(NB), bi.reshape(NB), ta.reshape(NB),
                      xs_bf, w1.astype(jnp.bfloat16), b1.reshape(E, 1, FF),
                      w2.astype(jnp.bfloat16), b2.reshape(E, 1, D),
                      roww.reshape(NPAD, 1))

    out = _combine(pos2[0], pos2[1], ys)
    return out.reshape(b, s, d)


# speedup vs baseline: 1.0027x; 1.0004x over previous
import functools

import jax
import jax.numpy as jnp
from jax import lax
from jax.experimental import pallas as pl
from jax.experimental.pallas import tpu as pltpu
from jax.experimental.pallas import tpu_sc as plsc

T = 2048
D = 1024
FF = 2048
E = 8
M = 256
NB = 24
NPAD = NB * M
NC = 2
NS = 16
NW = NC * NS
RPW = NPAD // NW
TPW = T // NW

_SQRT_2_OVER_PI = 0.7978845608028654
_GELU_C = 0.044715


def _gelu_tanh(h):
    return 0.5 * h * (1.0 + jnp.tanh(_SQRT_2_OVER_PI * (h + _GELU_C * h * h * h)))


def _router_plan_body(x_ref, gw_ref, pos_ref, w_ref, te_ref, bi_ref, ta_ref):
    xv = x_ref[...]
    gwv = gw_ref[...]
    lT = lax.dot_general(gwv, xv, (((1,), (1,)), ((), ())),
                         preferred_element_type=jnp.float32)
    esub = lax.broadcasted_iota(jnp.int32, (E, T), 0)
    m0 = jnp.max(lT, axis=0, keepdims=True)
    e0 = jnp.min(jnp.where(lT == m0, esub, E), axis=0, keepdims=True)
    lT1 = jnp.where(esub == e0, -1e30, lT)
    m1 = jnp.max(lT1, axis=0, keepdims=True)
    e1 = jnp.min(jnp.where(lT1 == m1, esub, E), axis=0, keepdims=True)
    dexp = jnp.exp(m1 - m0)
    w0 = 1.0 / (1.0 + dexp)
    w1c = dexp / (1.0 + dexp)

    h0 = (esub == e0).astype(jnp.float32)
    h1 = (esub == e1).astype(jnp.float32)

    lane = lax.broadcasted_iota(jnp.int32, (E, T), 1)

    def lane_cumsum(a):
        c = a
        s = 1
        while s < T:
            r = pltpu.roll(c, s, 1)
            c = c + jnp.where(lane >= s, r, 0.0)
            s *= 2
        return c

    c0 = lane_cumsum(h0)
    c1 = lane_cumsum(h1)
    cnt0 = jnp.sum(h0, axis=1, keepdims=True)
    cnt = cnt0 + jnp.sum(h1, axis=1, keepdims=True)

    tiles = jnp.floor((cnt + (M - 1)) * (1.0 / M))
    sub = lax.broadcasted_iota(jnp.int32, (E, 1), 0)
    tin = tiles
    s = 1
    while s < E:
        r = pltpu.roll(tin, s, 0)
        tin = tin + jnp.where(sub >= s, r, 0.0)
        s *= 2
    off = (tin - tiles) * M
    endv = tin * M
    nb_used = jnp.sum(tiles, axis=0, keepdims=True)

    p0 = jnp.sum(h0 * (off + c0 - 1.0), axis=0, keepdims=True)
    p1 = jnp.sum(h1 * (off + cnt0 + c1 - 1.0), axis=0, keepdims=True)
    pos_ref[...] = jnp.concatenate([p0, p1], axis=0).astype(jnp.int32)
    w_ref[...] = jnp.concatenate([w0, w1c], axis=0)

    tl = lax.broadcasted_iota(jnp.int32, (1, NB), 1).astype(jnp.float32)
    ts = tl * M
    last = (nb_used - 1.0) * M
    tsc = jnp.minimum(ts, last)
    tej = jnp.sum((tsc >= endv).astype(jnp.float32), axis=0, keepdims=True)
    act = ts < nb_used * M
    te_ref[...] = tej.astype(jnp.int32)
    bi_ref[...] = jnp.where(act, tl, nb_used - 1.0).astype(jnp.int32)
    ta_ref[...] = act.astype(jnp.int32)


def _router_plan(x, gw):
    return pl.pallas_call(
        _router_plan_body,
        out_shape=[
            jax.ShapeDtypeStruct((2, T), jnp.int32),
            jax.ShapeDtypeStruct((2, T), jnp.float32),
            jax.ShapeDtypeStruct((1, NB), jnp.int32),
            jax.ShapeDtypeStruct((1, NB), jnp.int32),
            jax.ShapeDtypeStruct((1, NB), jnp.int32),
        ],
    )(x, gw)


def _plan_scatter_body(pos_hbm, w_hbm, tok_hbm, rw_hbm, pos_v, w_v, tok_v, rw_v):
    c = lax.axis_index("c")
    s = lax.axis_index("s")
    wid = s * NC + c
    lo = wid * RPW
    pltpu.sync_copy(pos_hbm, pos_v)
    pltpu.sync_copy(w_hbm, w_v)
    zi = jnp.zeros((16,), jnp.int32)
    zf = jnp.zeros((16,), jnp.float32)
    for i in range(RPW // 16):
        tok_v[pl.ds(i * 16, 16)] = zi
        rw_v[pl.ds(i * 16, 16)] = zf
    lane = lax.iota(jnp.int32, 16)

    @plsc.parallel_loop(0, (2 * T) // 16, 1, unroll=4)
    def _(i):
        idx = pos_v[pl.ds(i * 16, 16)] - lo
        msk = (idx >= 0) & (idx < RPW)
        pv = lane + i * 16
        tv = pv & (T - 1)
        wv = w_v[pl.ds(i * 16, 16)]
        plsc.store_scatter(tok_v, [idx], tv, mask=msk)
        plsc.store_scatter(rw_v, [idx], wv, mask=msk)

    pltpu.sync_copy(tok_v, tok_hbm.at[pl.ds(lo, RPW)])
    pltpu.sync_copy(rw_v, rw_hbm.at[pl.ds(lo, RPW)])


def _plan_scatter(pos_flat, w_flat):
    mesh = plsc.VectorSubcoreMesh(core_axis_name="c", subcore_axis_name="s",
                                  num_cores=NC, num_subcores=NS)
    return pl.kernel(
        _plan_scatter_body,
        out_type=[
            jax.ShapeDtypeStruct((NPAD,), jnp.int32),
            jax.ShapeDtypeStruct((NPAD,), jnp.float32),
        ],
        mesh=mesh,
        compiler_params=pltpu.CompilerParams(needs_layout_passes=False),
        scratch_types=[
            pltpu.VMEM((2 * T,), jnp.int32),
            pltpu.VMEM((2 * T,), jnp.float32),
            pltpu.VMEM((RPW,), jnp.int32),
            pltpu.VMEM((RPW,), jnp.float32),
        ],
    )(pos_flat, w_flat)


_GCH = 16
_GDEPTH = 4


def _dispatch_gather_body(tok_hbm, xpk_hbm, xs_hbm, idx_v, b0, b1, b2, b3,
                          s0, s1, s2, s3):
    c = lax.axis_index("c")
    s = lax.axis_index("s")
    wid = s * NC + c
    base = wid * RPW
    pltpu.sync_copy(tok_hbm.at[pl.ds(base, RPW)], idx_v)
    bufs = (b0, b1, b2, b3)
    sems = (s0, s1, s2, s3)
    nch = RPW // _GCH

    def fire(ch):
        iv = idx_v[pl.ds(ch * _GCH, _GCH)]
        return pltpu.async_copy(xpk_hbm.at[iv], bufs[ch % _GDEPTH],
                                sems[ch % _GDEPTH])

    cp = [None] * nch
    for ch in range(min(_GDEPTH, nch)):
        cp[ch] = fire(ch)
    for ch in range(nch):
        cp[ch].wait()
        pltpu.sync_copy(bufs[ch % _GDEPTH],
                        xs_hbm.at[pl.ds(base + ch * _GCH, _GCH)])
        nxt = ch + _GDEPTH
        if nxt < nch:
            cp[nxt] = fire(nxt)


def _dispatch_gather(tok, x_pk):
    mesh = plsc.VectorSubcoreMesh(core_axis_name="c", subcore_axis_name="s",
                                  num_cores=NC, num_subcores=NS)
    return pl.kernel(
        _dispatch_gather_body,
        out_type=jax.ShapeDtypeStruct((NPAD, D // 2), jnp.int32),
        mesh=mesh,
        scratch_types=[
            pltpu.VMEM((RPW,), jnp.int32),
            pltpu.VMEM((_GCH, D // 2), jnp.int32),
            pltpu.VMEM((_GCH, D // 2), jnp.int32),
            pltpu.VMEM((_GCH, D // 2), jnp.int32),
            pltpu.VMEM((_GCH, D // 2), jnp.int32),
            pltpu.SemaphoreType.DMA,
            pltpu.SemaphoreType.DMA,
            pltpu.SemaphoreType.DMA,
            pltpu.SemaphoreType.DMA,
        ],
    )(tok, x_pk)


def _ffn_body(te_s, bi_s, ta_s, xs_ref, w1_ref, b1_ref, w2_ref, b2_ref,
              rw_ref, ys_ref):
    i = pl.program_id(0)

    @pl.when(ta_s[i] == 1)
    def _():
        xb = xs_ref[...]
        h = jnp.dot(xb, w1_ref[0], preferred_element_type=jnp.float32)
        h = h + b1_ref[0]
        g = _gelu_tanh(h) * rw_ref[...]
        y = jnp.dot(g.astype(jnp.bfloat16), w2_ref[0],
                    preferred_element_type=jnp.float32)
        ys_ref[...] = y + b2_ref[0]


def _grouped_ffn(te, bi, ta, xs_bf, w1b, b1, w2b, b2, rw_col):
    grid_spec = pltpu.PrefetchScalarGridSpec(
        num_scalar_prefetch=3,
        grid=(NB,),
        in_specs=[
            pl.BlockSpec((M, D), lambda i, te, bi, ta: (bi[i], 0)),
            pl.BlockSpec((1, D, FF), lambda i, te, bi, ta: (te[i], 0, 0)),
            pl.BlockSpec((1, 1, FF), lambda i, te, bi, ta: (te[i], 0, 0)),
            pl.BlockSpec((1, FF, D), lambda i, te, bi, ta: (te[i], 0, 0)),
            pl.BlockSpec((1, 1, D), lambda i, te, bi, ta: (te[i], 0, 0)),
            pl.BlockSpec((M, 1), lambda i, te, bi, ta: (bi[i], 0)),
        ],
        out_specs=pl.BlockSpec((M, D), lambda i, te, bi, ta: (bi[i], 0)),
    )
    return pl.pallas_call(
        _ffn_body,
        grid_spec=grid_spec,
        out_shape=jax.ShapeDtypeStruct((NPAD, D), jnp.float32),
    )(te, bi, ta, xs_bf, w1b, b1, w2b, b2, rw_col)


_CCH = 16


def _combine_body(p0_hbm, p1_hbm, ys_hbm, out_hbm, i0_v, i1_v, bufa0, bufa1,
                  bufb0, bufb1, sa0, sa1, sb0, sb1):
    c = lax.axis_index("c")
    s = lax.axis_index("s")
    gbase = (c * NS + s) * TPW
    pltpu.sync_copy(p0_hbm.at[pl.ds(gbase, TPW)], i0_v)
    pltpu.sync_copy(p1_hbm.at[pl.ds(gbase, TPW)], i1_v)
    bufa = (bufa0, bufa1)
    bufb = (bufb0, bufb1)
    sas = (sa0, sa1)
    sbs = (sb0, sb1)
    nrnd = TPW // _CCH

    def fire(rnd):
        pb = rnd % 2
        cb = rnd * _CCH
        cpa = pltpu.async_copy(ys_hbm.at[i0_v.at[pl.ds(cb, _CCH)]], bufa[pb],
                               sas[pb])
        cpb = pltpu.async_copy(ys_hbm.at[i1_v.at[pl.ds(cb, _CCH)]], bufb[pb],
                               sbs[pb])
        return cpa, cpb

    cps = fire(0)
    for rnd in range(nrnd):
        pb = rnd % 2
        cpa, cpb = cps
        cpa.wait()
        cpb.wait()
        if rnd + 1 < nrnd:
            cps = fire(rnd + 1)
        a = bufa[pb]
        b = bufb[pb]

        def radd(r, carry):
            for j in range(D // 16):
                sl = slice(j * 16, j * 16 + 16)
                a[r, sl] = a[r, sl] + b[r, sl]
            return carry

        lax.fori_loop(0, _CCH, radd, 0)
        pltpu.sync_copy(a, out_hbm.at[pl.ds(gbase + rnd * _CCH, _CCH)])


def _combine(pos0, pos1, ys):
    mesh = plsc.VectorSubcoreMesh(core_axis_name="c", subcore_axis_name="s",
                                  num_cores=NC, num_subcores=NS)
    return pl.kernel(
        _combine_body,
        out_type=jax.ShapeDtypeStruct((T, D), jnp.float32),
        mesh=mesh,
        compiler_params=pltpu.CompilerParams(needs_layout_passes=False),
        scratch_types=[
            pltpu.VMEM((TPW,), jnp.int32),
            pltpu.VMEM((TPW,), jnp.int32),
            pltpu.VMEM((_CCH, D), jnp.float32),
            pltpu.VMEM((_CCH, D), jnp.float32),
            pltpu.VMEM((_CCH, D), jnp.float32),
            pltpu.VMEM((_CCH, D), jnp.float32),
            pltpu.SemaphoreType.DMA,
            pltpu.SemaphoreType.DMA,
            pltpu.SemaphoreType.DMA,
            pltpu.SemaphoreType.DMA,
        ],
    )(pos0, pos1, ys)


def kernel(hidden_states, gate_w, w1, b1, w2, b2):
    b, s, d = hidden_states.shape
    x = hidden_states.reshape(T, D)

    pos2, wflat2, te, bi, ta = _router_plan(x, gate_w)
    tok, roww = _plan_scatter(pos2.reshape(2 * T), wflat2.reshape(2 * T))

    x_pk = lax.bitcast_convert_type(
        x.astype(jnp.bfloat16).reshape(T, D // 2, 2), jnp.int32)
    xs_pk = _dispatch_gather(tok, x_pk)
    xs_bf = lax.bitcast_convert_type(xs_pk, jnp.bfloat16).reshape(NPAD, D)

    ys = _grouped_ffn(te.reshape(NB), bi.reshape(NB), ta.reshape(NB),
                      xs_bf, w1.astype(jnp.bfloat16), b1.reshape(E, 1, FF),
                      w2.astype(jnp.bfloat16), b2.reshape(E, 1, D),
                      roww.reshape(NPAD, 1))

    out = _combine(pos2[0], pos2[1], ys)
    return out.reshape(b, s, d)


# speedup vs baseline: 1.5125x; 1.5085x over previous
import functools

import jax
import jax.numpy as jnp
from jax import lax
from jax.experimental import pallas as pl
from jax.experimental.pallas import tpu as pltpu
from jax.experimental.pallas import tpu_sc as plsc

T = 2048
D = 1024
FF = 2048
E = 8
M = 256
NB = 24
NPAD = NB * M
NC = 2
NS = 16
NW = NC * NS
RPW = NPAD // NW
TPW = T // NW

_SQRT_2_OVER_PI = 0.7978845608028654
_GELU_C = 0.044715


def _gelu_tanh(h):
    return 0.5 * h * (1.0 + jnp.tanh(_SQRT_2_OVER_PI * (h + _GELU_C * h * h * h)))


def _router_plan_body(x_ref, gw_ref, pos_ref, w_ref, te_ref, bi_ref, ta_ref):
    xv = x_ref[...]
    gwv = gw_ref[...]
    lT = lax.dot_general(gwv, xv, (((1,), (1,)), ((), ())),
                         preferred_element_type=jnp.float32)
    esub = lax.broadcasted_iota(jnp.int32, (E, T), 0)
    m0 = jnp.max(lT, axis=0, keepdims=True)
    e0 = jnp.min(jnp.where(lT == m0, esub, E), axis=0, keepdims=True)
    lT1 = jnp.where(esub == e0, -1e30, lT)
    m1 = jnp.max(lT1, axis=0, keepdims=True)
    e1 = jnp.min(jnp.where(lT1 == m1, esub, E), axis=0, keepdims=True)
    dexp = jnp.exp(m1 - m0)
    w0 = 1.0 / (1.0 + dexp)
    w1c = dexp / (1.0 + dexp)

    h0 = (esub == e0).astype(jnp.float32)
    h1 = (esub == e1).astype(jnp.float32)

    lane = lax.broadcasted_iota(jnp.int32, (E, T), 1)

    def lane_cumsum(a):
        c = a
        s = 1
        while s < T:
            r = pltpu.roll(c, s, 1)
            c = c + jnp.where(lane >= s, r, 0.0)
            s *= 2
        return c

    c0 = lane_cumsum(h0)
    c1 = lane_cumsum(h1)
    cnt0 = jnp.sum(h0, axis=1, keepdims=True)
    cnt = cnt0 + jnp.sum(h1, axis=1, keepdims=True)

    tiles = jnp.floor((cnt + (M - 1)) * (1.0 / M))
    sub = lax.broadcasted_iota(jnp.int32, (E, 1), 0)
    tin = tiles
    s = 1
    while s < E:
        r = pltpu.roll(tin, s, 0)
        tin = tin + jnp.where(sub >= s, r, 0.0)
        s *= 2
    off = (tin - tiles) * M
    endv = tin * M
    nb_used = jnp.sum(tiles, axis=0, keepdims=True)

    p0 = jnp.sum(h0 * (off + c0 - 1.0), axis=0, keepdims=True)
    p1 = jnp.sum(h1 * (off + cnt0 + c1 - 1.0), axis=0, keepdims=True)
    pos_ref[...] = jnp.concatenate([p0, p1], axis=0).astype(jnp.int32)
    w_ref[...] = jnp.concatenate([w0, w1c], axis=0)

    tl = lax.broadcasted_iota(jnp.int32, (1, NB), 1).astype(jnp.float32)
    ts = tl * M
    last = (nb_used - 1.0) * M
    tsc = jnp.minimum(ts, last)
    tej = jnp.sum((tsc >= endv).astype(jnp.float32), axis=0, keepdims=True)
    act = ts < nb_used * M
    te_ref[...] = tej.astype(jnp.int32)
    bi_ref[...] = jnp.where(act, tl, nb_used - 1.0).astype(jnp.int32)
    ta_ref[...] = act.astype(jnp.int32)


def _router_plan(x, gw):
    return pl.pallas_call(
        _router_plan_body,
        out_shape=[
            jax.ShapeDtypeStruct((2, T), jnp.int32),
            jax.ShapeDtypeStruct((2, T), jnp.float32),
            jax.ShapeDtypeStruct((1, NB), jnp.int32),
            jax.ShapeDtypeStruct((1, NB), jnp.int32),
            jax.ShapeDtypeStruct((1, NB), jnp.int32),
        ],
    )(x, gw)


def _plan_scatter_body(pos_hbm, w_hbm, tok_hbm, rw_hbm, pos_v, w_v, tok_v, rw_v):
    c = lax.axis_index("c")
    s = lax.axis_index("s")
    wid = s * NC + c
    lo = wid * RPW
    pltpu.sync_copy(pos_hbm, pos_v)
    pltpu.sync_copy(w_hbm, w_v)
    zi = jnp.zeros((16,), jnp.int32)
    zf = jnp.zeros((16,), jnp.float32)
    for i in range(RPW // 16):
        tok_v[pl.ds(i * 16, 16)] = zi
        rw_v[pl.ds(i * 16, 16)] = zf
    lane = lax.iota(jnp.int32, 16)

    @plsc.parallel_loop(0, (2 * T) // 16, 1, unroll=4)
    def _(i):
        idx = pos_v[pl.ds(i * 16, 16)] - lo
        msk = (idx >= 0) & (idx < RPW)
        pv = lane + i * 16
        tv = pv & (T - 1)
        wv = w_v[pl.ds(i * 16, 16)]
        plsc.store_scatter(tok_v, [idx], tv, mask=msk)
        plsc.store_scatter(rw_v, [idx], wv, mask=msk)

    pltpu.sync_copy(tok_v, tok_hbm.at[pl.ds(lo, RPW)])
    pltpu.sync_copy(rw_v, rw_hbm.at[pl.ds(lo, RPW)])


def _plan_scatter(pos_flat, w_flat):
    mesh = plsc.VectorSubcoreMesh(core_axis_name="c", subcore_axis_name="s",
                                  num_cores=NC, num_subcores=NS)
    return pl.kernel(
        _plan_scatter_body,
        out_type=[
            jax.ShapeDtypeStruct((NPAD,), jnp.int32),
            jax.ShapeDtypeStruct((NPAD,), jnp.float32),
        ],
        mesh=mesh,
        compiler_params=pltpu.CompilerParams(needs_layout_passes=False),
        scratch_types=[
            pltpu.VMEM((2 * T,), jnp.int32),
            pltpu.VMEM((2 * T,), jnp.float32),
            pltpu.VMEM((RPW,), jnp.int32),
            pltpu.VMEM((RPW,), jnp.float32),
        ],
    )(pos_flat, w_flat)


_GCH = 16
_GDEPTH = 4


def _dispatch_gather_body(tok_hbm, x_hbm, xs_hbm, idx_v, b0, b1, b2, b3,
                          s0, s1, s2, s3):
    c = lax.axis_index("c")
    s = lax.axis_index("s")
    wid = s * NC + c
    base = wid * RPW
    pltpu.sync_copy(tok_hbm.at[pl.ds(base, RPW)], idx_v)
    bufs = (b0, b1, b2, b3)
    sems = (s0, s1, s2, s3)
    nch = RPW // _GCH

    def fire(ch):
        iv = idx_v[pl.ds(ch * _GCH, _GCH)]
        return pltpu.async_copy(x_hbm.at[iv], bufs[ch % _GDEPTH],
                                sems[ch % _GDEPTH])

    cp = [None] * nch
    for ch in range(min(_GDEPTH, nch)):
        cp[ch] = fire(ch)
    for ch in range(nch):
        cp[ch].wait()
        pltpu.sync_copy(bufs[ch % _GDEPTH],
                        xs_hbm.at[pl.ds(base + ch * _GCH, _GCH)])
        nxt = ch + _GDEPTH
        if nxt < nch:
            cp[nxt] = fire(nxt)


def _dispatch_gather(tok, x):
    mesh = plsc.VectorSubcoreMesh(core_axis_name="c", subcore_axis_name="s",
                                  num_cores=NC, num_subcores=NS)
    return pl.kernel(
        _dispatch_gather_body,
        out_type=jax.ShapeDtypeStruct((NPAD, D), jnp.float32),
        mesh=mesh,
        scratch_types=[
            pltpu.VMEM((RPW,), jnp.int32),
            pltpu.VMEM((_GCH, D), jnp.float32),
            pltpu.VMEM((_GCH, D), jnp.float32),
            pltpu.VMEM((_GCH, D), jnp.float32),
            pltpu.VMEM((_GCH, D), jnp.float32),
            pltpu.SemaphoreType.DMA,
            pltpu.SemaphoreType.DMA,
            pltpu.SemaphoreType.DMA,
            pltpu.SemaphoreType.DMA,
        ],
    )(tok, x)


def _ffn_body(te_s, bi_s, ta_s, xs_ref, w1_ref, b1_ref, w2_ref, b2_ref,
              rw_ref, ys_ref):
    i = pl.program_id(0)

    @pl.when(ta_s[i] == 1)
    def _():
        xb = xs_ref[...].astype(jnp.bfloat16)
        h = jnp.dot(xb, w1_ref[0], preferred_element_type=jnp.float32)
        h = h + b1_ref[0]
        g = _gelu_tanh(h) * rw_ref[...]
        y = jnp.dot(g.astype(jnp.bfloat16), w2_ref[0],
                    preferred_element_type=jnp.float32)
        ys_ref[...] = y + b2_ref[0]


def _grouped_ffn(te, bi, ta, xs_bf, w1b, b1, w2b, b2, rw_col):
    grid_spec = pltpu.PrefetchScalarGridSpec(
        num_scalar_prefetch=3,
        grid=(NB,),
        in_specs=[
            pl.BlockSpec((M, D), lambda i, te, bi, ta: (bi[i], 0)),
            pl.BlockSpec((1, D, FF), lambda i, te, bi, ta: (te[i], 0, 0)),
            pl.BlockSpec((1, 1, FF), lambda i, te, bi, ta: (te[i], 0, 0)),
            pl.BlockSpec((1, FF, D), lambda i, te, bi, ta: (te[i], 0, 0)),
            pl.BlockSpec((1, 1, D), lambda i, te, bi, ta: (te[i], 0, 0)),
            pl.BlockSpec((M, 1), lambda i, te, bi, ta: (bi[i], 0)),
        ],
        out_specs=pl.BlockSpec((M, D), lambda i, te, bi, ta: (bi[i], 0)),
    )
    return pl.pallas_call(
        _ffn_body,
        grid_spec=grid_spec,
        out_shape=jax.ShapeDtypeStruct((NPAD, D), jnp.float32),
    )(te, bi, ta, xs_bf, w1b, b1, w2b, b2, rw_col)


_CCH = 16


def _combine_body(p0_hbm, p1_hbm, ys_hbm, out_hbm, i0_v, i1_v, bufa0, bufa1,
                  bufb0, bufb1, sa0, sa1, sb0, sb1):
    c = lax.axis_index("c")
    s = lax.axis_index("s")
    gbase = (c * NS + s) * TPW
    pltpu.sync_copy(p0_hbm.at[pl.ds(gbase, TPW)], i0_v)
    pltpu.sync_copy(p1_hbm.at[pl.ds(gbase, TPW)], i1_v)
    bufa = (bufa0, bufa1)
    bufb = (bufb0, bufb1)
    sas = (sa0, sa1)
    sbs = (sb0, sb1)
    nrnd = TPW // _CCH

    def fire(rnd):
        pb = rnd % 2
        cb = rnd * _CCH
        cpa = pltpu.async_copy(ys_hbm.at[i0_v.at[pl.ds(cb, _CCH)]], bufa[pb],
                               sas[pb])
        cpb = pltpu.async_copy(ys_hbm.at[i1_v.at[pl.ds(cb, _CCH)]], bufb[pb],
                               sbs[pb])
        return cpa, cpb

    cps = fire(0)
    for rnd in range(nrnd):
        pb = rnd % 2
        cpa, cpb = cps
        cpa.wait()
        cpb.wait()
        if rnd + 1 < nrnd:
            cps = fire(rnd + 1)
        a = bufa[pb]
        b = bufb[pb]

        def radd(r, carry):
            for j in range(D // 16):
                sl = slice(j * 16, j * 16 + 16)
                a[r, sl] = a[r, sl] + b[r, sl]
            return carry

        lax.fori_loop(0, _CCH, radd, 0)
        pltpu.sync_copy(a, out_hbm.at[pl.ds(gbase + rnd * _CCH, _CCH)])


def _combine(pos0, pos1, ys):
    mesh = plsc.VectorSubcoreMesh(core_axis_name="c", subcore_axis_name="s",
                                  num_cores=NC, num_subcores=NS)
    return pl.kernel(
        _combine_body,
        out_type=jax.ShapeDtypeStruct((T, D), jnp.float32),
        mesh=mesh,
        compiler_params=pltpu.CompilerParams(needs_layout_passes=False),
        scratch_types=[
            pltpu.VMEM((TPW,), jnp.int32),
            pltpu.VMEM((TPW,), jnp.int32),
            pltpu.VMEM((_CCH, D), jnp.float32),
            pltpu.VMEM((_CCH, D), jnp.float32),
            pltpu.VMEM((_CCH, D), jnp.float32),
            pltpu.VMEM((_CCH, D), jnp.float32),
            pltpu.SemaphoreType.DMA,
            pltpu.SemaphoreType.DMA,
            pltpu.SemaphoreType.DMA,
            pltpu.SemaphoreType.DMA,
        ],
    )(pos0, pos1, ys)


def kernel(hidden_states, gate_w, w1, b1, w2, b2):
    b, s, d = hidden_states.shape
    x = hidden_states.reshape(T, D)

    pos2, wflat2, te, bi, ta = _router_plan(x, gate_w)
    tok, roww = _plan_scatter(pos2.reshape(2 * T), wflat2.reshape(2 * T))

    xs = _dispatch_gather(tok, x)

    ys = _grouped_ffn(te.reshape(NB), bi.reshape(NB), ta.reshape(NB),
                      xs, w1.astype(jnp.bfloat16), b1.reshape(E, 1, FF),
                      w2.astype(jnp.bfloat16), b2.reshape(E, 1, D),
                      roww.reshape(NPAD, 1))

    out = _combine(pos2[0], pos2[1], ys)
    return out.reshape(b, s, d)


# speedup vs baseline: 1.7140x; 1.1332x over previous
import functools

import jax
import jax.numpy as jnp
from jax import lax
from jax.experimental import pallas as pl
from jax.experimental.pallas import tpu as pltpu
from jax.experimental.pallas import tpu_sc as plsc

T = 2048
D = 1024
FF = 2048
E = 8
M = 256
NB = 24
NPAD = NB * M
NC = 2
NS = 16
NW = NC * NS
RPW = NPAD // NW
TPW = T // NW

_SQRT_2_OVER_PI = 0.7978845608028654
_GELU_C = 0.044715


def _gelu_tanh(h):
    return 0.5 * h * (1.0 + jnp.tanh(_SQRT_2_OVER_PI * (h + _GELU_C * h * h * h)))


def _router_plan_body(x_ref, gw_ref, pos_ref, w_ref, te_ref, bi_ref, ta_ref,
                      xc_ref):
    xv = x_ref[...]
    xc_ref[...] = xv
    gwv = gw_ref[...]
    lT = lax.dot_general(gwv, xv, (((1,), (1,)), ((), ())),
                         preferred_element_type=jnp.float32)
    esub = lax.broadcasted_iota(jnp.int32, (E, T), 0)
    m0 = jnp.max(lT, axis=0, keepdims=True)
    e0 = jnp.min(jnp.where(lT == m0, esub, E), axis=0, keepdims=True)
    lT1 = jnp.where(esub == e0, -1e30, lT)
    m1 = jnp.max(lT1, axis=0, keepdims=True)
    e1 = jnp.min(jnp.where(lT1 == m1, esub, E), axis=0, keepdims=True)
    dexp = jnp.exp(m1 - m0)
    w0 = 1.0 / (1.0 + dexp)
    w1c = dexp / (1.0 + dexp)

    h0 = (esub == e0).astype(jnp.float32)
    h1 = (esub == e1).astype(jnp.float32)

    lane = lax.broadcasted_iota(jnp.int32, (E, T), 1)

    def lane_cumsum(a):
        c = a
        s = 1
        while s < T:
            r = pltpu.roll(c, s, 1)
            c = c + jnp.where(lane >= s, r, 0.0)
            s *= 2
        return c

    c0 = lane_cumsum(h0)
    c1 = lane_cumsum(h1)
    cnt0 = jnp.sum(h0, axis=1, keepdims=True)
    cnt = cnt0 + jnp.sum(h1, axis=1, keepdims=True)

    tiles = jnp.floor((cnt + (M - 1)) * (1.0 / M))
    sub = lax.broadcasted_iota(jnp.int32, (E, 1), 0)
    tin = tiles
    s = 1
    while s < E:
        r = pltpu.roll(tin, s, 0)
        tin = tin + jnp.where(sub >= s, r, 0.0)
        s *= 2
    off = (tin - tiles) * M
    endv = tin * M
    nb_used = jnp.sum(tiles, axis=0, keepdims=True)

    p0 = jnp.sum(h0 * (off + c0 - 1.0), axis=0, keepdims=True)
    p1 = jnp.sum(h1 * (off + cnt0 + c1 - 1.0), axis=0, keepdims=True)
    pos_ref[...] = jnp.concatenate([p0, p1], axis=0).astype(jnp.int32)
    w_ref[...] = jnp.concatenate([w0, w1c], axis=0)

    tl = lax.broadcasted_iota(jnp.int32, (1, NB), 1).astype(jnp.float32)
    ts = tl * M
    last = (nb_used - 1.0) * M
    tsc = jnp.minimum(ts, last)
    tej = jnp.sum((tsc >= endv).astype(jnp.float32), axis=0, keepdims=True)
    act = ts < nb_used * M
    te_ref[...] = tej.astype(jnp.int32)
    bi_ref[...] = jnp.where(act, tl, nb_used - 1.0).astype(jnp.int32)
    ta_ref[...] = act.astype(jnp.int32)


def _router_plan(x, gw):
    return pl.pallas_call(
        _router_plan_body,
        out_shape=[
            jax.ShapeDtypeStruct((2, T), jnp.int32),
            jax.ShapeDtypeStruct((2, T), jnp.float32),
            jax.ShapeDtypeStruct((1, NB), jnp.int32),
            jax.ShapeDtypeStruct((1, NB), jnp.int32),
            jax.ShapeDtypeStruct((1, NB), jnp.int32),
            jax.ShapeDtypeStruct((T, D), jnp.float32),
        ],
    )(x, gw)


def _plan_scatter_body(pos_hbm, w_hbm, tok_hbm, rw_hbm, pos_v, w_v, tok_v, rw_v):
    c = lax.axis_index("c")
    s = lax.axis_index("s")
    wid = s * NC + c
    lo = wid * RPW
    pltpu.sync_copy(pos_hbm, pos_v)
    pltpu.sync_copy(w_hbm, w_v)
    zi = jnp.zeros((16,), jnp.int32)
    zf = jnp.zeros((16,), jnp.float32)
    for i in range(RPW // 16):
        tok_v[pl.ds(i * 16, 16)] = zi
        rw_v[pl.ds(i * 16, 16)] = zf
    lane = lax.iota(jnp.int32, 16)

    @plsc.parallel_loop(0, (2 * T) // 16, 1, unroll=4)
    def _(i):
        idx = pos_v[pl.ds(i * 16, 16)] - lo
        msk = (idx >= 0) & (idx < RPW)
        pv = lane + i * 16
        tv = pv & (T - 1)
        wv = w_v[pl.ds(i * 16, 16)]
        plsc.store_scatter(tok_v, [idx], tv, mask=msk)
        plsc.store_scatter(rw_v, [idx], wv, mask=msk)

    pltpu.sync_copy(tok_v, tok_hbm.at[pl.ds(lo, RPW)])
    pltpu.sync_copy(rw_v, rw_hbm.at[pl.ds(lo, RPW)])


def _plan_scatter(pos_flat, w_flat):
    mesh = plsc.VectorSubcoreMesh(core_axis_name="c", subcore_axis_name="s",
                                  num_cores=NC, num_subcores=NS)
    return pl.kernel(
        _plan_scatter_body,
        out_type=[
            jax.ShapeDtypeStruct((NPAD,), jnp.int32),
            jax.ShapeDtypeStruct((NPAD,), jnp.float32),
        ],
        mesh=mesh,
        compiler_params=pltpu.CompilerParams(needs_layout_passes=False),
        scratch_types=[
            pltpu.VMEM((2 * T,), jnp.int32),
            pltpu.VMEM((2 * T,), jnp.float32),
            pltpu.VMEM((RPW,), jnp.int32),
            pltpu.VMEM((RPW,), jnp.float32),
        ],
    )(pos_flat, w_flat)


_GCH = 16
_GDEPTH = 4


def _dispatch_gather_body(tok_hbm, x_hbm, xs_hbm, idx_v, b0, b1, b2, b3,
                          s0, s1, s2, s3):
    c = lax.axis_index("c")
    s = lax.axis_index("s")
    wid = s * NC + c
    base = wid * RPW
    pltpu.sync_copy(tok_hbm.at[pl.ds(base, RPW)], idx_v)
    bufs = (b0, b1, b2, b3)
    sems = (s0, s1, s2, s3)
    nch = RPW // _GCH

    def fire(ch):
        return pltpu.async_copy(x_hbm.at[idx_v.at[pl.ds(ch * _GCH, _GCH)]],
                                bufs[ch % _GDEPTH], sems[ch % _GDEPTH])

    cp = [None] * nch
    for ch in range(min(_GDEPTH, nch)):
        cp[ch] = fire(ch)
    for ch in range(nch):
        cp[ch].wait()
        pltpu.sync_copy(bufs[ch % _GDEPTH],
                        xs_hbm.at[pl.ds(base + ch * _GCH, _GCH)])
        nxt = ch + _GDEPTH
        if nxt < nch:
            cp[nxt] = fire(nxt)


def _dispatch_gather(tok, x):
    mesh = plsc.VectorSubcoreMesh(core_axis_name="c", subcore_axis_name="s",
                                  num_cores=NC, num_subcores=NS)
    return pl.kernel(
        _dispatch_gather_body,
        out_type=jax.ShapeDtypeStruct((NPAD, D), jnp.float32),
        mesh=mesh,
        scratch_types=[
            pltpu.VMEM((RPW,), jnp.int32),
            pltpu.VMEM((_GCH, D), jnp.float32),
            pltpu.VMEM((_GCH, D), jnp.float32),
            pltpu.VMEM((_GCH, D), jnp.float32),
            pltpu.VMEM((_GCH, D), jnp.float32),
            pltpu.SemaphoreType.DMA,
            pltpu.SemaphoreType.DMA,
            pltpu.SemaphoreType.DMA,
            pltpu.SemaphoreType.DMA,
        ],
    )(tok, x)


def _ffn_body(te_s, bi_s, ta_s, xs_ref, w1_ref, b1_ref, w2_ref, b2_ref,
              rw_ref, ys_ref):
    i = pl.program_id(0)

    @pl.when(ta_s[i] == 1)
    def _():
        xb = xs_ref[...].astype(jnp.bfloat16)
        h = jnp.dot(xb, w1_ref[0].astype(jnp.bfloat16),
                    preferred_element_type=jnp.float32)
        h = h + b1_ref[0]
        g = _gelu_tanh(h) * rw_ref[...]
        y = jnp.dot(g.astype(jnp.bfloat16), w2_ref[0].astype(jnp.bfloat16),
                    preferred_element_type=jnp.float32)
        ys_ref[...] = y + b2_ref[0]


def _grouped_ffn(te, bi, ta, xs_bf, w1b, b1, w2b, b2, rw_col):
    grid_spec = pltpu.PrefetchScalarGridSpec(
        num_scalar_prefetch=3,
        grid=(NB,),
        in_specs=[
            pl.BlockSpec((M, D), lambda i, te, bi, ta: (bi[i], 0)),
            pl.BlockSpec((1, D, FF), lambda i, te, bi, ta: (te[i], 0, 0)),
            pl.BlockSpec((1, 1, FF), lambda i, te, bi, ta: (te[i], 0, 0)),
            pl.BlockSpec((1, FF, D), lambda i, te, bi, ta: (te[i], 0, 0)),
            pl.BlockSpec((1, 1, D), lambda i, te, bi, ta: (te[i], 0, 0)),
            pl.BlockSpec((M, 1), lambda i, te, bi, ta: (bi[i], 0)),
        ],
        out_specs=pl.BlockSpec((M, D), lambda i, te, bi, ta: (bi[i], 0)),
    )
    return pl.pallas_call(
        _ffn_body,
        grid_spec=grid_spec,
        out_shape=jax.ShapeDtypeStruct((NPAD, D), jnp.float32),
    )(te, bi, ta, xs_bf, w1b, b1, w2b, b2, rw_col)


_CCH = 16


def _combine_body(p0_hbm, p1_hbm, ys_hbm, out_hbm, i0_v, i1_v, bufa0, bufa1,
                  bufb0, bufb1, sa0, sa1, sb0, sb1):
    c = lax.axis_index("c")
    s = lax.axis_index("s")
    gbase = (c * NS + s) * TPW
    pltpu.sync_copy(p0_hbm.at[pl.ds(gbase, TPW)], i0_v)
    pltpu.sync_copy(p1_hbm.at[pl.ds(gbase, TPW)], i1_v)
    bufa = (bufa0, bufa1)
    bufb = (bufb0, bufb1)
    sas = (sa0, sa1)
    sbs = (sb0, sb1)
    nrnd = TPW // _CCH

    def fire(rnd):
        pb = rnd % 2
        cb = rnd * _CCH
        cpa = pltpu.async_copy(ys_hbm.at[i0_v.at[pl.ds(cb, _CCH)]], bufa[pb],
                               sas[pb])
        cpb = pltpu.async_copy(ys_hbm.at[i1_v.at[pl.ds(cb, _CCH)]], bufb[pb],
                               sbs[pb])
        return cpa, cpb

    cps = fire(0)
    for rnd in range(nrnd):
        pb = rnd % 2
        cpa, cpb = cps
        cpa.wait()
        cpb.wait()
        if rnd + 1 < nrnd:
            cps = fire(rnd + 1)
        a = bufa[pb]
        b = bufb[pb]

        def radd(r, carry):
            for j in range(D // 16):
                sl = slice(j * 16, j * 16 + 16)
                a[r, sl] = a[r, sl] + b[r, sl]
            return carry

        lax.fori_loop(0, _CCH, radd, 0)
        pltpu.sync_copy(a, out_hbm.at[pl.ds(gbase + rnd * _CCH, _CCH)])


def _combine(pos0, pos1, ys):
    mesh = plsc.VectorSubcoreMesh(core_axis_name="c", subcore_axis_name="s",
                                  num_cores=NC, num_subcores=NS)
    return pl.kernel(
        _combine_body,
        out_type=jax.ShapeDtypeStruct((T, D), jnp.float32),
        mesh=mesh,
        compiler_params=pltpu.CompilerParams(needs_layout_passes=False),
        scratch_types=[
            pltpu.VMEM((TPW,), jnp.int32),
            pltpu.VMEM((TPW,), jnp.int32),
            pltpu.VMEM((_CCH, D), jnp.float32),
            pltpu.VMEM((_CCH, D), jnp.float32),
            pltpu.VMEM((_CCH, D), jnp.float32),
            pltpu.VMEM((_CCH, D), jnp.float32),
            pltpu.SemaphoreType.DMA,
            pltpu.SemaphoreType.DMA,
            pltpu.SemaphoreType.DMA,
            pltpu.SemaphoreType.DMA,
        ],
    )(pos0, pos1, ys)


def kernel(hidden_states, gate_w, w1, b1, w2, b2):
    b, s, d = hidden_states.shape
    x = hidden_states.reshape(T, D)

    pos2, wflat2, te, bi, ta, xc = _router_plan(x, gate_w)
    tok, roww = _plan_scatter(pos2.reshape(2 * T), wflat2.reshape(2 * T))

    xs = _dispatch_gather(tok, xc)

    ys = _grouped_ffn(te.reshape(NB), bi.reshape(NB), ta.reshape(NB),
                      xs, w1, b1.reshape(E, 1, FF),
                      w2, b2.reshape(E, 1, D),
                      roww.reshape(NPAD, 1))

    out = _combine(pos2[0], pos2[1], ys)
    return out.reshape(b, s, d)


# speedup vs baseline: 2.9243x; 1.7062x over previous
import functools

import jax
import jax.numpy as jnp
from jax import lax
from jax.experimental import pallas as pl
from jax.experimental.pallas import tpu as pltpu
from jax.experimental.pallas import tpu_sc as plsc

T = 2048
D = 1024
FF = 2048
E = 8
M = 256
NB = 24
NPAD = NB * M
NC = 2
NS = 16
NW = NC * NS
RPW = NPAD // NW
TPW = T // NW

_SQRT_2_OVER_PI = 0.7978845608028654
_GELU_C = 0.044715


def _gelu_tanh(h):
    return 0.5 * h * (1.0 + jnp.tanh(_SQRT_2_OVER_PI * (h + _GELU_C * h * h * h)))


def _router_plan_body(x_ref, gw_ref, pos_ref, w_ref, te_ref, bi_ref, ta_ref,
                      xc_ref):
    xv = x_ref[...]
    xc_ref[...] = xv.astype(jnp.bfloat16)
    gwv = gw_ref[...]
    lT = lax.dot_general(gwv, xv, (((1,), (1,)), ((), ())),
                         preferred_element_type=jnp.float32)
    esub = lax.broadcasted_iota(jnp.int32, (E, T), 0)
    m0 = jnp.max(lT, axis=0, keepdims=True)
    e0 = jnp.min(jnp.where(lT == m0, esub, E), axis=0, keepdims=True)
    lT1 = jnp.where(esub == e0, -1e30, lT)
    m1 = jnp.max(lT1, axis=0, keepdims=True)
    e1 = jnp.min(jnp.where(lT1 == m1, esub, E), axis=0, keepdims=True)
    dexp = jnp.exp(m1 - m0)
    w0 = 1.0 / (1.0 + dexp)
    w1c = dexp / (1.0 + dexp)

    h0 = (esub == e0).astype(jnp.float32)
    h1 = (esub == e1).astype(jnp.float32)

    lane = lax.broadcasted_iota(jnp.int32, (E, T), 1)

    def lane_cumsum(a):
        c = a
        s = 1
        while s < T:
            r = pltpu.roll(c, s, 1)
            c = c + jnp.where(lane >= s, r, 0.0)
            s *= 2
        return c

    c0 = lane_cumsum(h0)
    c1 = lane_cumsum(h1)
    cnt0 = jnp.sum(h0, axis=1, keepdims=True)
    cnt = cnt0 + jnp.sum(h1, axis=1, keepdims=True)

    tiles = jnp.floor((cnt + (M - 1)) * (1.0 / M))
    sub = lax.broadcasted_iota(jnp.int32, (E, 1), 0)
    tin = tiles
    s = 1
    while s < E:
        r = pltpu.roll(tin, s, 0)
        tin = tin + jnp.where(sub >= s, r, 0.0)
        s *= 2
    off = (tin - tiles) * M
    endv = tin * M
    nb_used = jnp.sum(tiles, axis=0, keepdims=True)

    p0 = jnp.sum(h0 * (off + c0 - 1.0), axis=0, keepdims=True)
    p1 = jnp.sum(h1 * (off + cnt0 + c1 - 1.0), axis=0, keepdims=True)
    pos_ref[...] = jnp.concatenate([p0, p1], axis=0).astype(jnp.int32)
    w_ref[...] = jnp.concatenate([w0, w1c], axis=0)

    tl = lax.broadcasted_iota(jnp.int32, (1, NB), 1).astype(jnp.float32)
    ts = tl * M
    last = (nb_used - 1.0) * M
    tsc = jnp.minimum(ts, last)
    tej = jnp.sum((tsc >= endv).astype(jnp.float32), axis=0, keepdims=True)
    act = ts < nb_used * M
    te_ref[...] = tej.astype(jnp.int32)
    bi_ref[...] = jnp.where(act, tl, nb_used - 1.0).astype(jnp.int32)
    ta_ref[...] = act.astype(jnp.int32)


def _router_plan(x, gw):
    return pl.pallas_call(
        _router_plan_body,
        out_shape=[
            jax.ShapeDtypeStruct((2, T), jnp.int32),
            jax.ShapeDtypeStruct((2, T), jnp.float32),
            jax.ShapeDtypeStruct((1, NB), jnp.int32),
            jax.ShapeDtypeStruct((1, NB), jnp.int32),
            jax.ShapeDtypeStruct((1, NB), jnp.int32),
            jax.ShapeDtypeStruct((T, D), jnp.bfloat16),
        ],
    )(x, gw)


def _plan_scatter_body(pos_hbm, w_hbm, tok_hbm, rw_hbm, pos_v, w_v, tok_v, rw_v):
    c = lax.axis_index("c")
    s = lax.axis_index("s")
    wid = s * NC + c
    lo = wid * RPW
    pltpu.sync_copy(pos_hbm, pos_v)
    pltpu.sync_copy(w_hbm, w_v)
    zi = jnp.zeros((16,), jnp.int32)
    zf = jnp.zeros((16,), jnp.float32)
    for i in range(RPW // 16):
        tok_v[pl.ds(i * 16, 16)] = zi
        rw_v[pl.ds(i * 16, 16)] = zf
    lane = lax.iota(jnp.int32, 16)

    @plsc.parallel_loop(0, (2 * T) // 16, 1, unroll=4)
    def _(i):
        idx = pos_v[pl.ds(i * 16, 16)] - lo
        msk = (idx >= 0) & (idx < RPW)
        pv = lane + i * 16
        tv = pv & (T - 1)
        wv = w_v[pl.ds(i * 16, 16)]
        plsc.store_scatter(tok_v, [idx], tv, mask=msk)
        plsc.store_scatter(rw_v, [idx], wv, mask=msk)

    pltpu.sync_copy(tok_v, tok_hbm.at[pl.ds(lo, RPW)])
    pltpu.sync_copy(rw_v, rw_hbm.at[pl.ds(lo, RPW)])


def _plan_scatter(pos_flat, w_flat):
    mesh = plsc.VectorSubcoreMesh(core_axis_name="c", subcore_axis_name="s",
                                  num_cores=NC, num_subcores=NS)
    return pl.kernel(
        _plan_scatter_body,
        out_type=[
            jax.ShapeDtypeStruct((NPAD,), jnp.int32),
            jax.ShapeDtypeStruct((NPAD,), jnp.float32),
        ],
        mesh=mesh,
        compiler_params=pltpu.CompilerParams(needs_layout_passes=False),
        scratch_types=[
            pltpu.VMEM((2 * T,), jnp.int32),
            pltpu.VMEM((2 * T,), jnp.float32),
            pltpu.VMEM((RPW,), jnp.int32),
            pltpu.VMEM((RPW,), jnp.float32),
        ],
    )(pos_flat, w_flat)


_GCH = 16
_GDEPTH = 4


def _dispatch_gather_body(tok_hbm, x_hbm, xs_hbm, idx_v, b0, b1, b2, b3,
                          s0, s1, s2, s3):
    c = lax.axis_index("c")
    s = lax.axis_index("s")
    wid = s * NC + c
    base = wid * RPW
    pltpu.sync_copy(tok_hbm.at[pl.ds(base, RPW)], idx_v)
    bufs = (b0, b1, b2, b3)
    sems = (s0, s1, s2, s3)
    nch = RPW // _GCH

    def fire(ch):
        return pltpu.async_copy(x_hbm.at[idx_v.at[pl.ds(ch * _GCH, _GCH)]],
                                bufs[ch % _GDEPTH], sems[ch % _GDEPTH])

    cp = [None] * nch
    for ch in range(min(_GDEPTH, nch)):
        cp[ch] = fire(ch)
    for ch in range(nch):
        cp[ch].wait()
        pltpu.sync_copy(bufs[ch % _GDEPTH],
                        xs_hbm.at[pl.ds(base + ch * _GCH, _GCH)])
        nxt = ch + _GDEPTH
        if nxt < nch:
            cp[nxt] = fire(nxt)


def _dispatch_gather(tok, x):
    mesh = plsc.VectorSubcoreMesh(core_axis_name="c", subcore_axis_name="s",
                                  num_cores=NC, num_subcores=NS)
    return pl.kernel(
        _dispatch_gather_body,
        out_type=jax.ShapeDtypeStruct((NPAD, D), jnp.float32),
        mesh=mesh,
        scratch_types=[
            pltpu.VMEM((RPW,), jnp.int32),
            pltpu.VMEM((_GCH, D), jnp.float32),
            pltpu.VMEM((_GCH, D), jnp.float32),
            pltpu.VMEM((_GCH, D), jnp.float32),
            pltpu.VMEM((_GCH, D), jnp.float32),
            pltpu.SemaphoreType.DMA,
            pltpu.SemaphoreType.DMA,
            pltpu.SemaphoreType.DMA,
            pltpu.SemaphoreType.DMA,
        ],
    )(tok, x)


def _ffn_body(te_s, bi_s, ta_s, xc_ref, tok_ref, w1_ref, b1_ref, w2_ref,
              b2_ref, rw_ref, ys_ref):
    i = pl.program_id(0)

    @pl.when(ta_s[i] == 1)
    def _():
        tokb = tok_ref[...]
        lanes = lax.broadcasted_iota(jnp.int32, (M, T), 1)
        onehot = (tokb == lanes).astype(jnp.float32).astype(jnp.bfloat16)
        xb = jnp.dot(onehot, xc_ref[...],
                     preferred_element_type=jnp.float32).astype(jnp.bfloat16)
        h = jnp.dot(xb, w1_ref[0].astype(jnp.bfloat16),
                    preferred_element_type=jnp.float32)
        h = h + b1_ref[0]
        g = _gelu_tanh(h) * rw_ref[...]
        y = jnp.dot(g.astype(jnp.bfloat16), w2_ref[0].astype(jnp.bfloat16),
                    preferred_element_type=jnp.float32)
        ys_ref[...] = y + b2_ref[0]


def _grouped_ffn(te, bi, ta, xc, tok2, w1b, b1, w2b, b2, rw_col):
    grid_spec = pltpu.PrefetchScalarGridSpec(
        num_scalar_prefetch=3,
        grid=(NB,),
        in_specs=[
            pl.BlockSpec((T, D), lambda i, te, bi, ta: (0, 0)),
            pl.BlockSpec((M, 1), lambda i, te, bi, ta: (bi[i], 0)),
            pl.BlockSpec((1, D, FF), lambda i, te, bi, ta: (te[i], 0, 0)),
            pl.BlockSpec((1, 1, FF), lambda i, te, bi, ta: (te[i], 0, 0)),
            pl.BlockSpec((1, FF, D), lambda i, te, bi, ta: (te[i], 0, 0)),
            pl.BlockSpec((1, 1, D), lambda i, te, bi, ta: (te[i], 0, 0)),
            pl.BlockSpec((M, 1), lambda i, te, bi, ta: (bi[i], 0)),
        ],
        out_specs=pl.BlockSpec((M, D), lambda i, te, bi, ta: (bi[i], 0)),
    )
    return pl.pallas_call(
        _ffn_body,
        grid_spec=grid_spec,
        out_shape=jax.ShapeDtypeStruct((NPAD, D), jnp.float32),
    )(te, bi, ta, xc, tok2, w1b, b1, w2b, b2, rw_col)


_CCH = 16


def _combine_body(p0_hbm, p1_hbm, ys_hbm, out_hbm, i0_v, i1_v, bufa0, bufa1,
                  bufb0, bufb1, sa0, sa1, sb0, sb1):
    c = lax.axis_index("c")
    s = lax.axis_index("s")
    gbase = (c * NS + s) * TPW
    pltpu.sync_copy(p0_hbm.at[pl.ds(gbase, TPW)], i0_v)
    pltpu.sync_copy(p1_hbm.at[pl.ds(gbase, TPW)], i1_v)
    bufa = (bufa0, bufa1)
    bufb = (bufb0, bufb1)
    sas = (sa0, sa1)
    sbs = (sb0, sb1)
    nrnd = TPW // _CCH

    def fire(rnd):
        pb = rnd % 2
        cb = rnd * _CCH
        cpa = pltpu.async_copy(ys_hbm.at[i0_v.at[pl.ds(cb, _CCH)]], bufa[pb],
                               sas[pb])
        cpb = pltpu.async_copy(ys_hbm.at[i1_v.at[pl.ds(cb, _CCH)]], bufb[pb],
                               sbs[pb])
        return cpa, cpb

    cps = fire(0)
    for rnd in range(nrnd):
        pb = rnd % 2
        cpa, cpb = cps
        cpa.wait()
        cpb.wait()
        if rnd + 1 < nrnd:
            cps = fire(rnd + 1)
        a = bufa[pb]
        b = bufb[pb]

        def radd(r, carry):
            for j in range(D // 16):
                sl = slice(j * 16, j * 16 + 16)
                a[r, sl] = a[r, sl] + b[r, sl]
            return carry

        lax.fori_loop(0, _CCH, radd, 0)
        pltpu.sync_copy(a, out_hbm.at[pl.ds(gbase + rnd * _CCH, _CCH)])


def _combine(pos0, pos1, ys):
    mesh = plsc.VectorSubcoreMesh(core_axis_name="c", subcore_axis_name="s",
                                  num_cores=NC, num_subcores=NS)
    return pl.kernel(
        _combine_body,
        out_type=jax.ShapeDtypeStruct((T, D), jnp.float32),
        mesh=mesh,
        compiler_params=pltpu.CompilerParams(needs_layout_passes=False),
        scratch_types=[
            pltpu.VMEM((TPW,), jnp.int32),
            pltpu.VMEM((TPW,), jnp.int32),
            pltpu.VMEM((_CCH, D), jnp.float32),
            pltpu.VMEM((_CCH, D), jnp.float32),
            pltpu.VMEM((_CCH, D), jnp.float32),
            pltpu.VMEM((_CCH, D), jnp.float32),
            pltpu.SemaphoreType.DMA,
            pltpu.SemaphoreType.DMA,
            pltpu.SemaphoreType.DMA,
            pltpu.SemaphoreType.DMA,
        ],
    )(pos0, pos1, ys)


def kernel(hidden_states, gate_w, w1, b1, w2, b2):
    b, s, d = hidden_states.shape
    x = hidden_states.reshape(T, D)

    pos2, wflat2, te, bi, ta, xc = _router_plan(x, gate_w)
    tok, roww = _plan_scatter(pos2.reshape(2 * T), wflat2.reshape(2 * T))

    ys = _grouped_ffn(te.reshape(NB), bi.reshape(NB), ta.reshape(NB),
                      xc, tok.reshape(NPAD, 1), w1, b1.reshape(E, 1, FF),
                      w2, b2.reshape(E, 1, D),
                      roww.reshape(NPAD, 1))

    out = _combine(pos2[0], pos2[1], ys)
    return out.reshape(b, s, d)
